# baseline (speedup 1.0000x reference)
import functools

import jax
import jax.numpy as jnp
from jax import lax
from jax.experimental import pallas as pl
from jax.experimental.pallas import tpu as pltpu
from jax.experimental.pallas import tpu_sc as plsc

VOCAB = 1000000
HIDDEN = 64
BATCH = 16384
HIST = 50

B = BATCH * HIST
NC, NS = 2, 16
NW = NC * NS
LANE = 128
CBLK = BATCH // LANE
NBLOCK = HIST * CBLK
BLKW = NBLOCK // NW
BPW = BLKW * LANE
NBUF = 2
TG = 4
NGRP = BLKW // TG
TBUF = 2

_mesh = plsc.VectorSubcoreMesh(core_axis_name="c", subcore_axis_name="s")


@functools.partial(
    pl.kernel,
    mesh=_mesh,
    out_type=jax.ShapeDtypeStruct(
        (HIST, HIDDEN // 8, CBLK, 8 * LANE), jnp.float32
    ),
    scratch_types=[
        pltpu.VMEM((BPW,), jnp.int32),
        [pltpu.VMEM((LANE, HIDDEN), jnp.float32) for _ in range(NBUF)],
        [pltpu.VMEM((HIDDEN // 8, TG, 8 * LANE), jnp.float32) for _ in range(TBUF)],
        [pltpu.SemaphoreType.DMA for _ in range(NBUF)],
        [pltpu.SemaphoreType.DMA for _ in range(TBUF)],
    ],
    compiler_params=pltpu.CompilerParams(
        use_tc_tiling_on_sc=False, needs_layout_passes=False
    ),
)
def _gather_kernel(idx_hbm, table_hbm, out_hbm, idx_v, rows, tbs, sem_in, sem_out):
    wid = lax.axis_index("s") * NC + lax.axis_index("c")
    base = wid * BLKW

    lanes16 = lax.iota(jnp.int32, 16)
    dt_vecs = [(lanes16 + dg * 16) // 8 for dg in range(HIDDEN // 16)]
    sl_vecs = [
        lax.rem(lanes16 + dg * 16, 8) * LANE for dg in range(HIDDEN // 16)
    ]
    qb_vecs = [jnp.zeros((16,), jnp.int32) + qb for qb in range(TG)]

    def fire(g, j):
        pltpu.async_copy(
            table_hbm.at[idx_v.at[pl.ds(g * LANE, LANE)]], rows[j], sem_in[j]
        )

    def wait_gather(j):
        pltpu.make_async_copy(
            table_hbm.at[idx_v.at[pl.ds(0, LANE)]], rows[j], sem_in[j]
        ).wait()

    def transpose(j, t, qb):
        @plsc.parallel_loop(0, LANE, unroll=8)
        def _row(l):
            lbc = jnp.zeros((16,), jnp.int32) + l
            for dg in range(HIDDEN // 16):
                v = rows[j][l, pl.ds(dg * 16, 16)]
                plsc.store_scatter(
                    tbs[t],
                    [dt_vecs[dg], qb_vecs[qb], sl_vecs[dg] + lbc],
                    v,
                )

    def start_writeback(q, t):
        beta = base + q * TG
        h = beta // CBLK
        c = lax.rem(beta, CBLK)
        pltpu.async_copy(tbs[t], out_hbm.at[h, :, pl.ds(c, TG), :], sem_out[t])

    def wait_writeback(t):
        pltpu.make_async_copy(
            tbs[t], out_hbm.at[0, :, pl.ds(0, TG), :], sem_out[t]
        ).wait()

    pltpu.sync_copy(idx_hbm.at[pl.ds(wid * BPW, BPW)], idx_v)

    fire(0, 0)
    fire(1, 1)

    for q in range(TBUF):
        t = q % TBUF
        for qb in range(TG):
            g = q * TG + qb
            wait_gather(qb % NBUF)
            transpose(qb % NBUF, t, qb)
            fire(g + 2, qb % NBUF)
        start_writeback(q, t)

    def body(qp, carry):
        for t in range(TBUF):
            q_ = qp * TBUF + t
            wait_writeback(t)
            for qb in range(TG):
                g = q_ * TG + qb
                wait_gather(qb % NBUF)
                transpose(qb % NBUF, t, qb)
                fire(g + 2, qb % NBUF)
            start_writeback(q_, t)
        return carry

    lax.fori_loop(1, NGRP // TBUF - 1, body, 0)

    for q in (NGRP - 2, NGRP - 1):
        t = q % TBUF
        wait_writeback(t)
        for qb in range(TG):
            g = q * TG + qb
            wait_gather(qb % NBUF)
            transpose(qb % NBUF, t, qb)
            if g + 2 < BLKW:
                fire(g + 2, qb % NBUF)
        start_writeback(q, t)

    for t in range(TBUF):
        wait_writeback(t)


def kernel(input, weight):
    idx = input.T.reshape(-1)
    out4 = _gather_kernel(idx, weight)
    out5 = out4.reshape(HIST, HIDDEN // 8, CBLK, 8, LANE)
    return out5.transpose(2, 4, 0, 1, 3).reshape(BATCH, HIST, HIDDEN)


# speedup vs baseline: 1.0992x; 1.0992x over previous
import functools

import jax
import jax.numpy as jnp
from jax import lax
from jax.experimental import pallas as pl
from jax.experimental.pallas import tpu as pltpu
from jax.experimental.pallas import tpu_sc as plsc

VOCAB = 1000000
HIDDEN = 64
BATCH = 16384
HIST = 50

B = BATCH * HIST
NC, NS = 2, 16
NW = NC * NS
BPW = B // NW
CHUNK = 256
NCHUNK = BPW // CHUNK
NBUF = 4

_mesh = plsc.VectorSubcoreMesh(core_axis_name="c", subcore_axis_name="s")


@functools.partial(
    pl.kernel,
    mesh=_mesh,
    out_type=jax.ShapeDtypeStruct((B, HIDDEN), jnp.float32),
    scratch_types=[
        pltpu.VMEM((BPW,), jnp.int32),
        [pltpu.VMEM((CHUNK, HIDDEN), jnp.float32) for _ in range(NBUF)],
        [pltpu.SemaphoreType.DMA for _ in range(NBUF)],
        [pltpu.SemaphoreType.DMA for _ in range(NBUF)],
    ],
    compiler_params=pltpu.CompilerParams(use_tc_tiling_on_sc=False),
)
def _gather_kernel(idx_hbm, table_hbm, out_hbm, idx_v, rows, sem_in, sem_out):
    wid = lax.axis_index("s") * NC + lax.axis_index("c")
    base = wid * BPW

    def fire(g, j):
        pltpu.async_copy(
            table_hbm.at[idx_v.at[pl.ds(g * CHUNK, CHUNK)]], rows[j], sem_in[j]
        )

    def wait_gather(j):
        pltpu.make_async_copy(
            table_hbm.at[idx_v.at[pl.ds(0, CHUNK)]], rows[j], sem_in[j]
        ).wait()

    def start_writeback(g, j):
        pltpu.async_copy(
            rows[j], out_hbm.at[pl.ds(base + g * CHUNK, CHUNK)], sem_out[j]
        )

    def wait_writeback(j):
        pltpu.make_async_copy(
            rows[j], out_hbm.at[pl.ds(base, CHUNK)], sem_out[j]
        ).wait()

    pltpu.sync_copy(idx_hbm.at[pl.ds(base, BPW)], idx_v)

    fire(0, 0)
    fire(1, 1)

    for j in range(NBUF):
        wait_gather(j)
        start_writeback(j, j)
        if j + 2 < NBUF:
            fire(j + 2, j + 2)
        else:
            wait_writeback((j + 2) % NBUF)
            fire(j + 2, (j + 2) % NBUF)

    def body(go, carry):
        for j in range(NBUF):
            g = go * NBUF + j
            wait_gather(j)
            start_writeback(g, j)
            wait_writeback((j + 2) % NBUF)
            fire(g + 2, (j + 2) % NBUF)
        return carry

    lax.fori_loop(1, NCHUNK // NBUF - 1, body, 0)

    for j in range(NBUF):
        g = NCHUNK - NBUF + j
        wait_gather(j)
        start_writeback(g, j)
        if j < 2:
            wait_writeback((j + 2) % NBUF)
            fire(g + 2, (j + 2) % NBUF)

    for j in range(NBUF):
        wait_writeback(j)


def kernel(input, weight):
    idx = input.T.reshape(-1)
    out = _gather_kernel(idx, weight)
    return out.reshape(HIST, BATCH, HIDDEN).transpose(1, 0, 2)
